# baseline (speedup 1.0000x reference)
import functools

import jax
import jax.numpy as jnp
from jax import lax
from jax.experimental import pallas as pl
from jax.experimental.pallas import tpu as pltpu
from jax.experimental.pallas import tpu_sc as plsc


def _fold_body(emb_ref, w_ref, b_ref, out_ref):
    t = lax.dot_general(
        emb_ref[...], w_ref[...],
        (((1,), (1,)), ((), ())),
        preferred_element_type=jnp.float32,
        precision=lax.Precision.HIGHEST,
    )
    out_ref[...] = t + b_ref[...]


def _fold_table(emb, W, b2d):
    V, _ = emb.shape
    Cp = W.shape[0]
    return pl.pallas_call(
        _fold_body,
        out_shape=jax.ShapeDtypeStruct((V, Cp), jnp.float32),
    )(emb, W, b2d)


def _gather_transposed(table1d, idxw, NB, L, C):
    VW = table1d.shape[0]
    info = plsc.get_sparse_core_info()
    nc, ns = info.num_cores, info.num_subcores
    nw = nc * ns
    Lt = (L + 7) // 8
    n_rows = C * Lt * 8 * nw
    mesh = plsc.VectorSubcoreMesh(core_axis_name="c", subcore_axis_name="s")

    @functools.partial(
        pl.kernel,
        mesh=mesh,
        out_type=jax.ShapeDtypeStruct((n_rows, 128), jnp.float32),
        scratch_types=[
            pltpu.VMEM((VW,), jnp.float32),
            pltpu.VMEM((Lt * 8, 128), jnp.int32),
            pltpu.VMEM((C, 8, 128), jnp.float32),
            pltpu.SemaphoreType.DMA,
        ],
        compiler_params=pltpu.CompilerParams(
            use_tc_tiling_on_sc=False, needs_layout_passes=False),
    )
    def k(table_hbm, idx_hbm, out_hbm, tab_v, idx_v, pack_v, sem_o):
        wid = lax.axis_index("s") * nc + lax.axis_index("c")
        pltpu.sync_copy(table_hbm, tab_v)
        pltpu.sync_copy(idx_hbm.at[pl.ds(wid * L, L)],
                        idx_v.at[pl.ds(0, L)])
        for t in range(L, Lt * 8):
            pltpu.sync_copy(idx_hbm.at[pl.ds(wid * L, 1)],
                            idx_v.at[pl.ds(t, 1)])

        def body(jt, carry):
            for jl in range(8):
                j = jt * 8 + jl
                for h in range(8):
                    idx16 = idx_v[j, pl.ds(h * 16, 16)]
                    rows32 = idx16 * 32
                    for c in range(C):
                        v = plsc.load_gather(tab_v, [rows32 + c])
                        pack_v[c, jl, pl.ds(h * 16, 16)] = v
            for c in range(C):
                r0 = (c * Lt + jt) * (8 * nw) + wid * 8
                pltpu.async_copy(pack_v.at[c], out_hbm.at[pl.ds(r0, 8)],
                                 sem_o)
            for c in range(C):
                r0 = (c * Lt + jt) * (8 * nw) + wid * 8
                pltpu.make_async_copy(
                    pack_v.at[c], out_hbm.at[pl.ds(r0, 8)], sem_o).wait()
            return carry

        lax.fori_loop(0, Lt, body, 0)

    return k(table1d, idxw)


def kernel(x, emb, W, b):
    NB, L = x.shape
    C = W.shape[0]
    Cp = 32
    Wp = jnp.pad(W.astype(jnp.float32), ((0, Cp - C), (0, 0)))
    bp = jnp.pad(b.astype(jnp.float32), (0, Cp - C)).reshape(1, Cp)
    tab = _fold_table(emb, Wp, bp)

    idxw = (x.T.astype(jnp.int32)
            .reshape(L, NB // 128, 128)
            .transpose(1, 0, 2)
            .reshape(NB // 128 * L, 128))
    raw = _gather_transposed(tab.reshape(-1), idxw, NB, L, C)

    Lt = (L + 7) // 8
    r = raw.reshape(C, Lt, NB // 128, 8, 128)
    t = r.transpose(2, 4, 1, 3, 0)
    f = t.reshape(NB, Lt * 8, C)
    return f[:, :L, :]


# speedup vs baseline: 1.4910x; 1.4910x over previous
import functools

import jax
import jax.numpy as jnp
from jax import lax
from jax.experimental import pallas as pl
from jax.experimental.pallas import tpu as pltpu
from jax.experimental.pallas import tpu_sc as plsc


def _fold_body(emb_ref, w_ref, b_ref, out_ref):
    t = lax.dot_general(
        emb_ref[...], w_ref[...],
        (((1,), (1,)), ((), ())),
        preferred_element_type=jnp.float32,
        precision=lax.Precision.HIGHEST,
    )
    out_ref[...] = t + b_ref[...]


def _fold_table(emb, W, b2d):
    V, _ = emb.shape
    Cp = W.shape[0]
    return pl.pallas_call(
        _fold_body,
        out_shape=jax.ShapeDtypeStruct((V, Cp), jnp.float32),
    )(emb, W, b2d)


def _gather_transposed(table1d, idxw, NB, L, C):
    VW = table1d.shape[0]
    V = VW // 32
    info = plsc.get_sparse_core_info()
    nc, ns = info.num_cores, info.num_subcores
    nw = nc * ns
    Lt = (L + 7) // 8
    n_rows = C * Lt * 8 * nw
    mesh = plsc.VectorSubcoreMesh(core_axis_name="c", subcore_axis_name="s")

    @functools.partial(
        pl.kernel,
        mesh=mesh,
        out_type=jax.ShapeDtypeStruct((n_rows, 128), jnp.float32),
        scratch_types=[
            pltpu.VMEM((VW,), jnp.float32),
            pltpu.VMEM((Lt * 8, 128), jnp.int32),
            pltpu.VMEM((C, 8, 128), jnp.float32),
            pltpu.SemaphoreType.DMA,
        ],
        compiler_params=pltpu.CompilerParams(
            use_tc_tiling_on_sc=False, needs_layout_passes=False),
    )
    def k(table_hbm, idx_hbm, out_hbm, tab_v, idx_v, pack_v, sem_o):
        wid = lax.axis_index("s") * nc + lax.axis_index("c")
        pltpu.sync_copy(table_hbm, tab_v)
        pltpu.sync_copy(idx_hbm.at[pl.ds(wid * L, L)],
                        idx_v.at[pl.ds(0, L)])
        for t in range(L, Lt * 8):
            pltpu.sync_copy(idx_hbm.at[pl.ds(wid * L, 1)],
                            idx_v.at[pl.ds(t, 1)])

        def body(jt, carry):
            for jl in range(8):
                j = jt * 8 + jl
                for h in range(8):
                    idx16 = idx_v[j, pl.ds(h * 16, 16)]
                    for c in range(C):
                        v = plsc.load_gather(tab_v, [idx16 + c * V])
                        pack_v[c, jl, pl.ds(h * 16, 16)] = v
            for c in range(C):
                r0 = (c * Lt + jt) * (8 * nw) + wid * 8
                pltpu.async_copy(pack_v.at[c], out_hbm.at[pl.ds(r0, 8)],
                                 sem_o)
            for c in range(C):
                r0 = (c * Lt + jt) * (8 * nw) + wid * 8
                pltpu.make_async_copy(
                    pack_v.at[c], out_hbm.at[pl.ds(r0, 8)], sem_o).wait()
            return carry

        lax.fori_loop(0, Lt, body, 0)

    return k(table1d, idxw)


def kernel(x, emb, W, b):
    NB, L = x.shape
    C = W.shape[0]
    Cp = 32
    Wp = jnp.pad(W.astype(jnp.float32), ((0, Cp - C), (0, 0)))
    bp = jnp.pad(b.astype(jnp.float32), (0, Cp - C)).reshape(1, Cp)
    tab = _fold_table(emb, Wp, bp)

    idxw = (x.T.astype(jnp.int32)
            .reshape(L, NB // 128, 128)
            .transpose(1, 0, 2)
            .reshape(NB // 128 * L, 128))
    raw = _gather_transposed(tab.T.reshape(-1), idxw, NB, L, C)

    Lt = (L + 7) // 8
    r = raw.reshape(C, Lt, NB // 128, 8, 128)
    t = r.transpose(2, 4, 1, 3, 0)
    f = t.reshape(NB, Lt * 8, C)
    return f[:, :L, :]
